# Initial kernel scaffold; baseline (speedup 1.0000x reference)
import jax
import jax.numpy as jnp
from jax.experimental import pallas as pl


def kernel(x, edge_index, W1, att_src1, att_dst1, b1, W2, att_src2, att_dst2, b2):
    raise NotImplementedError("write your pallas kernel here")


# speedup vs baseline: 38.8733x; 38.8733x over previous
import functools
import jax
import jax.numpy as jnp
from jax import lax
from jax.experimental import pallas as pl
from jax.experimental.pallas import tpu as pltpu
from jax.experimental.pallas import tpu_sc as plsc

N = 10000
E = 320000
IN_CH = 128
HID = 16
HEADS = 8
D1 = HEADS * HID
D2 = 64

NP = 10240
NC = 2
NS = 16
NW = NC * NS
CHUNK = 128
EPT = 10240
E_PAD = NW * EPT
N_CHUNKS = EPT // CHUNK
ROWS_PER_SUB = NP // NS

NEG_BIG = -1.0e30
BLK = 256


def _leaky(x):
    return jnp.where(x > 0, x, 0.2 * x)


def _tc1_body(x_ref, w_ref, atts_ref, attd_ref, sel_ref,
              h_ref, ta_ref, tb_ref, wself_ref):
    i = pl.program_id(0)
    h = jnp.dot(x_ref[...], w_ref[...], preferred_element_type=jnp.float32)
    h_ref[...] = h
    a_s = jnp.dot(h * atts_ref[...], sel_ref[...],
                  preferred_element_type=jnp.float32)
    a_d = jnp.dot(h * attd_ref[...], sel_ref[...],
                  preferred_element_type=jnp.float32)
    rows = i * BLK + lax.broadcasted_iota(jnp.int32, (BLK, HEADS), 0)
    valid = rows < N
    a_s_m = jnp.where(valid, a_s, NEG_BIG)
    a_d_m = jnp.where(valid, a_d, NEG_BIG)
    ta_ref[...] = jnp.concatenate([a_s_m, a_s_m], axis=1)
    tb_ref[...] = jnp.concatenate([a_d_m, a_d_m], axis=1)
    ws = jnp.exp(_leaky(a_s + a_d))
    wself_ref[...] = jnp.where(valid, ws, 0.0)


def _tc1(xp, W1, atts_row, attd_row, sel):
    grid = (NP // BLK,)
    return pl.pallas_call(
        _tc1_body,
        grid=grid,
        in_specs=[
            pl.BlockSpec((BLK, IN_CH), lambda i: (i, 0)),
            pl.BlockSpec((IN_CH, D1), lambda i: (0, 0)),
            pl.BlockSpec((1, D1), lambda i: (0, 0)),
            pl.BlockSpec((1, D1), lambda i: (0, 0)),
            pl.BlockSpec((D1, HEADS), lambda i: (0, 0)),
        ],
        out_specs=[
            pl.BlockSpec((BLK, D1), lambda i: (i, 0)),
            pl.BlockSpec((BLK, 2 * HEADS), lambda i: (i, 0)),
            pl.BlockSpec((BLK, 2 * HEADS), lambda i: (i, 0)),
            pl.BlockSpec((BLK, HEADS), lambda i: (i, 0)),
        ],
        out_shape=[
            jax.ShapeDtypeStruct((NP, D1), jnp.float32),
            jax.ShapeDtypeStruct((NP, 2 * HEADS), jnp.float32),
            jax.ShapeDtypeStruct((NP, 2 * HEADS), jnp.float32),
            jax.ShapeDtypeStruct((NP, HEADS), jnp.float32),
        ],
    )(xp, W1, atts_row, attd_row, sel)


def _make_sc_edge(D, multi_head):
    n_vec = D // 16
    mesh = plsc.VectorSubcoreMesh(core_axis_name="c", subcore_axis_name="s")

    @functools.partial(
        pl.kernel,
        mesh=mesh,
        compiler_params=pltpu.CompilerParams(use_tc_tiling_on_sc=False),
        out_type=[
            jax.ShapeDtypeStruct((NC, NP, D), jnp.float32),
            jax.ShapeDtypeStruct((NC, NP, 16), jnp.float32),
        ],
        scratch_types=dict(
            acc_sh=pltpu.VMEM_SHARED((NP, D), jnp.float32),
            den_sh=pltpu.VMEM_SHARED((NP, 16), jnp.float32),
            sidx_v=pltpu.VMEM((CHUNK,), jnp.int32),
            didx_v=pltpu.VMEM((CHUNK,), jnp.int32),
            ta_v=pltpu.VMEM((CHUNK, 16), jnp.float32),
            tb_v=pltpu.VMEM((CHUNK, 16), jnp.float32),
            w_v=pltpu.VMEM((CHUNK, 16), jnp.float32),
            h_v=pltpu.VMEM((CHUNK, D), jnp.float32),
            sem_h=pltpu.SemaphoreType.DMA,
            sem_a=pltpu.SemaphoreType.DMA,
            sem_b=pltpu.SemaphoreType.DMA,
        ),
    )
    def k(h_hbm, ta_hbm, tb_hbm, src_hbm, dst_hbm, zacc_hbm, zden_hbm,
          outp_hbm, denp_hbm,
          acc_sh, den_sh, sidx_v, didx_v, ta_v, tb_v, w_v, h_v,
          sem_h, sem_a, sem_b):
        cid = lax.axis_index("c")
        sid = lax.axis_index("s")
        wid = cid * NS + sid

        r0 = sid * ROWS_PER_SUB
        pltpu.sync_copy(zacc_hbm.at[pl.ds(r0, ROWS_PER_SUB), :],
                        acc_sh.at[pl.ds(r0, ROWS_PER_SUB), :])
        pltpu.sync_copy(zden_hbm.at[pl.ds(r0, ROWS_PER_SUB), :],
                        den_sh.at[pl.ds(r0, ROWS_PER_SUB), :])
        plsc.subcore_barrier()

        def chunk_body(g, carry):
            base = wid * EPT + g * CHUNK
            pltpu.sync_copy(src_hbm.at[pl.ds(base, CHUNK)], sidx_v)
            pltpu.sync_copy(dst_hbm.at[pl.ds(base, CHUNK)], didx_v)
            cp_h = pltpu.async_copy(h_hbm.at[sidx_v], h_v, sem_h)
            cp_a = pltpu.async_copy(ta_hbm.at[sidx_v], ta_v, sem_a)
            cp_b = pltpu.async_copy(tb_hbm.at[didx_v], tb_v, sem_b)
            cp_a.wait()
            cp_b.wait()

            def w_body(c, carry2):
                alpha = ta_v[c, :] + tb_v[c, :]
                w_v[c, :] = jnp.exp(jnp.where(alpha > 0, alpha, 0.2 * alpha))
                return carry2

            lax.fori_loop(0, CHUNK, w_body, 0, unroll=4)
            pltpu.sync_copy(w_v, den_sh.at[didx_v], add=True)
            cp_h.wait()

            def mul_body(c, carry2):
                wrow = w_v[c, :]
                for k2 in range(n_vec):
                    hidx = k2 if multi_head else 0
                    wsc = wrow[hidx]
                    h_v[c, pl.ds(k2 * 16, 16)] = (
                        h_v[c, pl.ds(k2 * 16, 16)] * wsc)
                return carry2

            lax.fori_loop(0, CHUNK, mul_body, 0, unroll=2)
            pltpu.sync_copy(h_v, acc_sh.at[didx_v], add=True)
            return carry

        lax.fori_loop(0, N_CHUNKS, chunk_body, 0)

        plsc.subcore_barrier()
        pltpu.sync_copy(acc_sh.at[pl.ds(r0, ROWS_PER_SUB), :],
                        outp_hbm.at[cid, pl.ds(r0, ROWS_PER_SUB), :])
        pltpu.sync_copy(den_sh.at[pl.ds(r0, ROWS_PER_SUB), :],
                        denp_hbm.at[cid, pl.ds(r0, ROWS_PER_SUB), :])

    return k


_sc_edge_l1 = _make_sc_edge(D1, True)
_sc_edge_l2 = _make_sc_edge(D2, False)


def _tc2_body(o0_ref, o1_ref, d0_ref, d1_ref, h1_ref, wself_ref, selT_ref,
              b1_ref, w2_ref, atts2_ref, attd2_ref,
              h2_ref, ta2_ref, tb2_ref, wself2_ref):
    i = pl.program_id(0)
    wself = wself_ref[...]
    wrep = jnp.dot(wself, selT_ref[...],
                   preferred_element_type=jnp.float32)
    num = o0_ref[...] + o1_ref[...] + h1_ref[...] * wrep
    den8 = d0_ref[...][:, :HEADS] + d1_ref[...][:, :HEADS] + wself
    den = jnp.dot(den8, selT_ref[...], preferred_element_type=jnp.float32)
    g = jnp.maximum(num / den + b1_ref[...], 0.0)
    h2 = jnp.dot(g, w2_ref[...], preferred_element_type=jnp.float32)
    h2_ref[...] = h2
    t_s = jnp.sum(h2 * atts2_ref[...], axis=1, keepdims=True)
    t_d = jnp.sum(h2 * attd2_ref[...], axis=1, keepdims=True)
    rows = i * BLK + lax.broadcasted_iota(jnp.int32, (BLK, 16), 0)
    valid = rows < N
    ta2_ref[...] = jnp.where(valid, jnp.broadcast_to(t_s, (BLK, 16)), NEG_BIG)
    tb2_ref[...] = jnp.where(valid, jnp.broadcast_to(t_d, (BLK, 16)), NEG_BIG)
    ws2 = jnp.exp(_leaky(t_s + t_d))
    wself2_ref[...] = jnp.where(valid, jnp.broadcast_to(ws2, (BLK, 16)), 0.0)


def _tc2(o0, o1, d0, d1, h1, wself, selT, b1row, W2, atts2, attd2):
    grid = (NP // BLK,)
    return pl.pallas_call(
        _tc2_body,
        grid=grid,
        in_specs=[
            pl.BlockSpec((BLK, D1), lambda i: (i, 0)),
            pl.BlockSpec((BLK, D1), lambda i: (i, 0)),
            pl.BlockSpec((BLK, 16), lambda i: (i, 0)),
            pl.BlockSpec((BLK, 16), lambda i: (i, 0)),
            pl.BlockSpec((BLK, D1), lambda i: (i, 0)),
            pl.BlockSpec((BLK, HEADS), lambda i: (i, 0)),
            pl.BlockSpec((HEADS, D1), lambda i: (0, 0)),
            pl.BlockSpec((1, D1), lambda i: (0, 0)),
            pl.BlockSpec((D1, D2), lambda i: (0, 0)),
            pl.BlockSpec((1, D2), lambda i: (0, 0)),
            pl.BlockSpec((1, D2), lambda i: (0, 0)),
        ],
        out_specs=[
            pl.BlockSpec((BLK, D2), lambda i: (i, 0)),
            pl.BlockSpec((BLK, 16), lambda i: (i, 0)),
            pl.BlockSpec((BLK, 16), lambda i: (i, 0)),
            pl.BlockSpec((BLK, 16), lambda i: (i, 0)),
        ],
        out_shape=[
            jax.ShapeDtypeStruct((NP, D2), jnp.float32),
            jax.ShapeDtypeStruct((NP, 16), jnp.float32),
            jax.ShapeDtypeStruct((NP, 16), jnp.float32),
            jax.ShapeDtypeStruct((NP, 16), jnp.float32),
        ],
    )(o0, o1, d0, d1, h1, wself, selT, b1row, W2, atts2, attd2)


def _tc3_body(p0_ref, p1_ref, q0_ref, q1_ref, h2_ref, wself2_ref, b2_ref,
              out_ref):
    ws = wself2_ref[...][:, 0:1]
    den = q0_ref[...][:, 0:1] + q1_ref[...][:, 0:1] + ws
    num = p0_ref[...] + p1_ref[...] + h2_ref[...] * ws
    out_ref[...] = num / den + b2_ref[...]


def _tc3(p0, p1, q0, q1, h2, wself2, b2row):
    grid = (NP // BLK,)
    return pl.pallas_call(
        _tc3_body,
        grid=grid,
        in_specs=[
            pl.BlockSpec((BLK, D2), lambda i: (i, 0)),
            pl.BlockSpec((BLK, D2), lambda i: (i, 0)),
            pl.BlockSpec((BLK, 16), lambda i: (i, 0)),
            pl.BlockSpec((BLK, 16), lambda i: (i, 0)),
            pl.BlockSpec((BLK, D2), lambda i: (i, 0)),
            pl.BlockSpec((BLK, 16), lambda i: (i, 0)),
            pl.BlockSpec((1, D2), lambda i: (0, 0)),
        ],
        out_specs=pl.BlockSpec((BLK, D2), lambda i: (i, 0)),
        out_shape=jax.ShapeDtypeStruct((NP, D2), jnp.float32),
    )(p0, p1, q0, q1, h2, wself2, b2row)


@jax.jit
def _run(x, edge_index, W1, att_src1, att_dst1, b1, W2, att_src2, att_dst2,
         b2):
    f32 = jnp.float32
    xp = jnp.zeros((NP, IN_CH), f32).at[:N].set(x)
    srcp = jnp.full((E_PAD,), N, jnp.int32).at[:E].set(edge_index[0])
    dstp = jnp.full((E_PAD,), N, jnp.int32).at[:E].set(edge_index[1])

    col = jnp.arange(D1) // HID
    sel = (col[:, None] == jnp.arange(HEADS)[None, :]).astype(f32)
    selT = sel.T

    atts_row = att_src1.reshape(1, D1)
    attd_row = att_dst1.reshape(1, D1)
    h1, ta1, tb1, wself1 = _tc1(xp, W1, atts_row, attd_row, sel)

    zacc1 = jnp.zeros((NP, D1), f32)
    zden = jnp.zeros((NP, 16), f32)
    outp1, denp1 = _sc_edge_l1(h1, ta1, tb1, srcp, dstp, zacc1, zden)

    h2, ta2, tb2, wself2 = _tc2(
        outp1[0], outp1[1], denp1[0], denp1[1], h1, wself1, selT,
        b1.reshape(1, D1), W2, att_src2.reshape(1, D2),
        att_dst2.reshape(1, D2))

    zacc2 = jnp.zeros((NP, D2), f32)
    outp2, denp2 = _sc_edge_l2(h2, ta2, tb2, srcp, dstp, zacc2, zden)

    out = _tc3(outp2[0], outp2[1], denp2[0], denp2[1], h2, wself2,
               b2.reshape(1, D2))
    return out[:N]


def kernel(x, edge_index, W1, att_src1, att_dst1, b1, W2, att_src2, att_dst2,
           b2):
    return _run(x, edge_index, W1, att_src1, att_dst1, b1, W2, att_src2,
                att_dst2, b2)


# speedup vs baseline: 49.3449x; 1.2694x over previous
import functools
import jax
import jax.numpy as jnp
from jax import lax
from jax.experimental import pallas as pl
from jax.experimental.pallas import tpu as pltpu
from jax.experimental.pallas import tpu_sc as plsc

N = 10000
E = 320000
IN_CH = 128
HID = 16
HEADS = 8
D1 = HEADS * HID
D2 = 64

NP = 10240
NC = 2
NS = 16
NW = NC * NS
CHUNK = 80
EPT = 10240
E_PAD = NW * EPT
N_CHUNKS = EPT // CHUNK
ROWS_PER_SUB = NP // NS

NEG_BIG = -1.0e30
BLK = 256


def _leaky(x):
    return jnp.maximum(x, 0.2 * x)


def _tc1_body(x_ref, w_ref, atts_ref, attd_ref, sel_ref,
              h_ref, ta_ref, tb_ref, wself_ref):
    i = pl.program_id(0)
    h = jnp.dot(x_ref[...], w_ref[...], preferred_element_type=jnp.float32)
    h_ref[...] = h
    a_s = jnp.dot(h * atts_ref[...], sel_ref[...],
                  preferred_element_type=jnp.float32)
    a_d = jnp.dot(h * attd_ref[...], sel_ref[...],
                  preferred_element_type=jnp.float32)
    rows = i * BLK + lax.broadcasted_iota(jnp.int32, (BLK, HEADS), 0)
    valid = rows < N
    a_s_m = jnp.where(valid, a_s, NEG_BIG)
    a_d_m = jnp.where(valid, a_d, NEG_BIG)
    ta_ref[...] = jnp.concatenate([a_s_m, a_s_m], axis=1)
    tb_ref[...] = jnp.concatenate([a_d_m, a_d_m], axis=1)
    ws = jnp.exp(_leaky(a_s + a_d))
    wself_ref[...] = jnp.where(valid, ws, 0.0)


def _tc1(xp, W1, atts_row, attd_row, sel):
    grid = (NP // BLK,)
    return pl.pallas_call(
        _tc1_body,
        grid=grid,
        in_specs=[
            pl.BlockSpec((BLK, IN_CH), lambda i: (i, 0)),
            pl.BlockSpec((IN_CH, D1), lambda i: (0, 0)),
            pl.BlockSpec((1, D1), lambda i: (0, 0)),
            pl.BlockSpec((1, D1), lambda i: (0, 0)),
            pl.BlockSpec((D1, HEADS), lambda i: (0, 0)),
        ],
        out_specs=[
            pl.BlockSpec((BLK, D1), lambda i: (i, 0)),
            pl.BlockSpec((BLK, 2 * HEADS), lambda i: (i, 0)),
            pl.BlockSpec((BLK, 2 * HEADS), lambda i: (i, 0)),
            pl.BlockSpec((BLK, HEADS), lambda i: (i, 0)),
        ],
        out_shape=[
            jax.ShapeDtypeStruct((NP, D1), jnp.float32),
            jax.ShapeDtypeStruct((NP, 2 * HEADS), jnp.float32),
            jax.ShapeDtypeStruct((NP, 2 * HEADS), jnp.float32),
            jax.ShapeDtypeStruct((NP, HEADS), jnp.float32),
        ],
    )(xp, W1, atts_row, attd_row, sel)


def _make_sc_edge(D, multi_head):
    n_vec = D // 16
    mesh = plsc.VectorSubcoreMesh(core_axis_name="c", subcore_axis_name="s")

    @functools.partial(
        pl.kernel,
        mesh=mesh,
        compiler_params=pltpu.CompilerParams(use_tc_tiling_on_sc=False),
        out_type=[
            jax.ShapeDtypeStruct((NC, NP, D), jnp.float32),
            jax.ShapeDtypeStruct((NC, NP, 16), jnp.float32),
        ],
        scratch_types=dict(
            acc_sh=pltpu.VMEM_SHARED((NP, D), jnp.float32),
            den_sh=pltpu.VMEM_SHARED((NP, 16), jnp.float32),
            sidx_v=[pltpu.VMEM((CHUNK,), jnp.int32)] * 2,
            didx_v=[pltpu.VMEM((CHUNK,), jnp.int32)] * 2,
            ta_v=[pltpu.VMEM((CHUNK, 16), jnp.float32)] * 2,
            tb_v=[pltpu.VMEM((CHUNK, 16), jnp.float32)] * 2,
            w_v=pltpu.VMEM((CHUNK, 16), jnp.float32),
            h_v=[pltpu.VMEM((CHUNK, D), jnp.float32)] * 2,
            sem_h=[pltpu.SemaphoreType.DMA] * 2,
            sem_a=[pltpu.SemaphoreType.DMA] * 2,
            sem_b=[pltpu.SemaphoreType.DMA] * 2,
            sem_si=[pltpu.SemaphoreType.DMA] * 2,
            sem_di=[pltpu.SemaphoreType.DMA] * 2,
        ),
    )
    def k(h_hbm, ta_hbm, tb_hbm, src_hbm, dst_hbm, zacc_hbm, zden_hbm,
          outp_hbm, denp_hbm,
          acc_sh, den_sh, sidx_v, didx_v, ta_v, tb_v, w_v, h_v,
          sem_h, sem_a, sem_b, sem_si, sem_di):
        cid = lax.axis_index("c")
        sid = lax.axis_index("s")
        wid = cid * NS + sid

        r0 = sid * ROWS_PER_SUB
        pltpu.sync_copy(zacc_hbm.at[pl.ds(r0, ROWS_PER_SUB), :],
                        acc_sh.at[pl.ds(r0, ROWS_PER_SUB), :])
        pltpu.sync_copy(zden_hbm.at[pl.ds(r0, ROWS_PER_SUB), :],
                        den_sh.at[pl.ds(r0, ROWS_PER_SUB), :])
        plsc.subcore_barrier()

        ebase = wid * EPT

        def issue_idx(c, p):
            d1 = pltpu.async_copy(src_hbm.at[pl.ds(ebase + c * CHUNK, CHUNK)],
                                  sidx_v[p], sem_si[p])
            d2 = pltpu.async_copy(dst_hbm.at[pl.ds(ebase + c * CHUNK, CHUNK)],
                                  didx_v[p], sem_di[p])
            return (d1, d2)

        def issue_gathers(p):
            d1 = pltpu.async_copy(h_hbm.at[sidx_v[p]], h_v[p], sem_h[p])
            d2 = pltpu.async_copy(ta_hbm.at[sidx_v[p]], ta_v[p], sem_a[p])
            d3 = pltpu.async_copy(tb_hbm.at[didx_v[p]], tb_v[p], sem_b[p])
            return (d1, d2, d3)

        def wait_all(descs):
            for d in descs:
                d.wait()

        def process(p):
            def w_body(c, carry2):
                alpha = ta_v[p][c, :] + tb_v[p][c, :]
                w_v[c, :] = jnp.exp(jnp.maximum(alpha, 0.2 * alpha))
                return carry2

            lax.fori_loop(0, CHUNK, w_body, 0, unroll=4)
            pltpu.sync_copy(w_v, den_sh.at[didx_v[p]], add=True)

            def mul_body(c, carry2):
                wrow = w_v[c, :]
                for k2 in range(n_vec):
                    hidx = k2 if multi_head else 0
                    wsc = wrow[hidx]
                    h_v[p][c, pl.ds(k2 * 16, 16)] = (
                        h_v[p][c, pl.ds(k2 * 16, 16)] * wsc)
                return carry2

            lax.fori_loop(0, CHUNK, mul_body, 0, unroll=2)
            pltpu.sync_copy(h_v[p], acc_sh.at[didx_v[p]], add=True)

        pltpu.sync_copy(src_hbm.at[pl.ds(ebase, CHUNK)], sidx_v[0])
        pltpu.sync_copy(dst_hbm.at[pl.ds(ebase, CHUNK)], didx_v[0])
        wait_all(issue_gathers(0))
        pltpu.sync_copy(src_hbm.at[pl.ds(ebase + CHUNK, CHUNK)], sidx_v[1])
        pltpu.sync_copy(dst_hbm.at[pl.ds(ebase + CHUNK, CHUNK)], didx_v[1])

        def pair_body(g, carry):
            c0 = 2 * g
            g1 = issue_gathers(1)
            process(0)
            i0 = issue_idx(c0 + 2, 0)
            wait_all(g1)
            wait_all(i0)
            g0 = issue_gathers(0)
            process(1)
            i1 = issue_idx(c0 + 3, 1)
            wait_all(g0)
            wait_all(i1)
            return carry

        lax.fori_loop(0, N_CHUNKS // 2, pair_body, 0)

        plsc.subcore_barrier()
        pltpu.sync_copy(acc_sh.at[pl.ds(r0, ROWS_PER_SUB), :],
                        outp_hbm.at[cid, pl.ds(r0, ROWS_PER_SUB), :])
        pltpu.sync_copy(den_sh.at[pl.ds(r0, ROWS_PER_SUB), :],
                        denp_hbm.at[cid, pl.ds(r0, ROWS_PER_SUB), :])

    return k


_sc_edge_l1 = _make_sc_edge(D1, True)
_sc_edge_l2 = _make_sc_edge(D2, False)


def _tc2_body(o0_ref, o1_ref, d0_ref, d1_ref, h1_ref, wself_ref, selT_ref,
              b1_ref, w2_ref, atts2_ref, attd2_ref,
              h2_ref, ta2_ref, tb2_ref, wself2_ref):
    i = pl.program_id(0)
    wself = wself_ref[...]
    wrep = jnp.dot(wself, selT_ref[...],
                   preferred_element_type=jnp.float32)
    num = o0_ref[...] + o1_ref[...] + h1_ref[...] * wrep
    den8 = d0_ref[...][:, :HEADS] + d1_ref[...][:, :HEADS] + wself
    den = jnp.dot(den8, selT_ref[...], preferred_element_type=jnp.float32)
    g = jnp.maximum(num / den + b1_ref[...], 0.0)
    h2 = jnp.dot(g, w2_ref[...], preferred_element_type=jnp.float32)
    h2_ref[...] = h2
    t_s = jnp.sum(h2 * atts2_ref[...], axis=1, keepdims=True)
    t_d = jnp.sum(h2 * attd2_ref[...], axis=1, keepdims=True)
    rows = i * BLK + lax.broadcasted_iota(jnp.int32, (BLK, 16), 0)
    valid = rows < N
    ta2_ref[...] = jnp.where(valid, jnp.broadcast_to(t_s, (BLK, 16)), NEG_BIG)
    tb2_ref[...] = jnp.where(valid, jnp.broadcast_to(t_d, (BLK, 16)), NEG_BIG)
    ws2 = jnp.exp(_leaky(t_s + t_d))
    wself2_ref[...] = jnp.where(valid, jnp.broadcast_to(ws2, (BLK, 16)), 0.0)


def _tc2(o0, o1, d0, d1, h1, wself, selT, b1row, W2, atts2, attd2):
    grid = (NP // BLK,)
    return pl.pallas_call(
        _tc2_body,
        grid=grid,
        in_specs=[
            pl.BlockSpec((BLK, D1), lambda i: (i, 0)),
            pl.BlockSpec((BLK, D1), lambda i: (i, 0)),
            pl.BlockSpec((BLK, 16), lambda i: (i, 0)),
            pl.BlockSpec((BLK, 16), lambda i: (i, 0)),
            pl.BlockSpec((BLK, D1), lambda i: (i, 0)),
            pl.BlockSpec((BLK, HEADS), lambda i: (i, 0)),
            pl.BlockSpec((HEADS, D1), lambda i: (0, 0)),
            pl.BlockSpec((1, D1), lambda i: (0, 0)),
            pl.BlockSpec((D1, D2), lambda i: (0, 0)),
            pl.BlockSpec((1, D2), lambda i: (0, 0)),
            pl.BlockSpec((1, D2), lambda i: (0, 0)),
        ],
        out_specs=[
            pl.BlockSpec((BLK, D2), lambda i: (i, 0)),
            pl.BlockSpec((BLK, 16), lambda i: (i, 0)),
            pl.BlockSpec((BLK, 16), lambda i: (i, 0)),
            pl.BlockSpec((BLK, 16), lambda i: (i, 0)),
        ],
        out_shape=[
            jax.ShapeDtypeStruct((NP, D2), jnp.float32),
            jax.ShapeDtypeStruct((NP, 16), jnp.float32),
            jax.ShapeDtypeStruct((NP, 16), jnp.float32),
            jax.ShapeDtypeStruct((NP, 16), jnp.float32),
        ],
    )(o0, o1, d0, d1, h1, wself, selT, b1row, W2, atts2, attd2)


def _tc3_body(p0_ref, p1_ref, q0_ref, q1_ref, h2_ref, wself2_ref, b2_ref,
              out_ref):
    ws = wself2_ref[...][:, 0:1]
    den = q0_ref[...][:, 0:1] + q1_ref[...][:, 0:1] + ws
    num = p0_ref[...] + p1_ref[...] + h2_ref[...] * ws
    out_ref[...] = num / den + b2_ref[...]


def _tc3(p0, p1, q0, q1, h2, wself2, b2row):
    grid = (NP // BLK,)
    return pl.pallas_call(
        _tc3_body,
        grid=grid,
        in_specs=[
            pl.BlockSpec((BLK, D2), lambda i: (i, 0)),
            pl.BlockSpec((BLK, D2), lambda i: (i, 0)),
            pl.BlockSpec((BLK, 16), lambda i: (i, 0)),
            pl.BlockSpec((BLK, 16), lambda i: (i, 0)),
            pl.BlockSpec((BLK, D2), lambda i: (i, 0)),
            pl.BlockSpec((BLK, 16), lambda i: (i, 0)),
            pl.BlockSpec((1, D2), lambda i: (0, 0)),
        ],
        out_specs=pl.BlockSpec((BLK, D2), lambda i: (i, 0)),
        out_shape=jax.ShapeDtypeStruct((NP, D2), jnp.float32),
    )(p0, p1, q0, q1, h2, wself2, b2row)


@jax.jit
def _run(x, edge_index, W1, att_src1, att_dst1, b1, W2, att_src2, att_dst2,
         b2):
    f32 = jnp.float32
    xp = jnp.zeros((NP, IN_CH), f32).at[:N].set(x)
    srcp = jnp.full((E_PAD + 2 * CHUNK,), N, jnp.int32).at[:E].set(
        edge_index[0])
    dstp = jnp.full((E_PAD + 2 * CHUNK,), N, jnp.int32).at[:E].set(
        edge_index[1])

    col = jnp.arange(D1) // HID
    sel = (col[:, None] == jnp.arange(HEADS)[None, :]).astype(f32)
    selT = sel.T

    atts_row = att_src1.reshape(1, D1)
    attd_row = att_dst1.reshape(1, D1)
    h1, ta1, tb1, wself1 = _tc1(xp, W1, atts_row, attd_row, sel)

    zacc1 = jnp.zeros((NP, D1), f32)
    zden = jnp.zeros((NP, 16), f32)
    outp1, denp1 = _sc_edge_l1(h1, ta1, tb1, srcp, dstp, zacc1, zden)

    h2, ta2, tb2, wself2 = _tc2(
        outp1[0], outp1[1], denp1[0], denp1[1], h1, wself1, selT,
        b1.reshape(1, D1), W2, att_src2.reshape(1, D2),
        att_dst2.reshape(1, D2))

    zacc2 = jnp.zeros((NP, D2), f32)
    outp2, denp2 = _sc_edge_l2(h2, ta2, tb2, srcp, dstp, zacc2, zden)

    out = _tc3(outp2[0], outp2[1], denp2[0], denp2[1], h2, wself2,
               b2.reshape(1, D2))
    return out[:N]


def kernel(x, edge_index, W1, att_src1, att_dst1, b1, W2, att_src2, att_dst2,
           b2):
    return _run(x, edge_index, W1, att_src1, att_dst1, b1, W2, att_src2,
                att_dst2, b2)
